# baseline (speedup 1.0000x reference)
import functools

import jax
import jax.numpy as jnp
from jax.experimental import pallas as pl
from jax.experimental.pallas import tpu as pltpu

_GAMMA = 20
_LAMBDA_INV = 25.0
_NEG_BIG = -3.4e38
_BIG_I = 2 ** 30


def _fold_rows(v, idx, irides):
    while v.shape[0] > 8:
        h = v.shape[0] // 2
        take = v[:h] >= v[h:]
        v = jnp.where(take, v[:h], v[h:])
        idx = jnp.where(take, idx[:h], idx[h:])
        irides = [jnp.where(take, r[:h], r[h:]) for r in irides]
    vw = jnp.max(v, axis=0, keepdims=True)
    iw = jnp.min(jnp.where(v == vw, idx, _BIG_I), axis=0, keepdims=True)
    sel = idx == iw
    ir = [jnp.max(jnp.where(sel, r, -1), axis=0, keepdims=True)
          for r in irides]
    return vw, iw, ir


def _body(vidr_ref, vidc_ref, z_ref, loss_ref, cos_ref,
          sbest_ref, sbj_ref, snorm_ref, bias_ref, *, bb_per_prog, nprog,
          B, P, D):
    i = pl.program_id(0)

    @pl.when(i == 0)
    def _():
        vidr = vidr_ref[...]
        vidc = vidc_ref[...]
        rids = jax.lax.broadcasted_iota(jnp.int32, (P, P), 0)
        cids = jax.lax.broadcasted_iota(jnp.int32, (P, P), 1)
        mask = (vidc != vidr) & (rids != cids)
        bias_ref[...] = jnp.where(mask, 0.0, _NEG_BIG)

    bias = bias_ref[...]
    row_ids = jax.lax.broadcasted_iota(jnp.int32, (P, 128), 0)
    diag128 = (jax.lax.broadcasted_iota(jnp.int32, (128, 128), 0)
               == jax.lax.broadcasted_iota(jnp.int32, (128, 128), 1))
    base = i * bb_per_prog
    for bb in range(bb_per_prog):
        zb = z_ref[bb]
        sim = jax.lax.dot_general(
            zb, zb, (((1,), (1,)), ((), ())),
            preferred_element_type=jnp.float32)
        vs, js, norms = [], [], []
        for t in range(P // 128):
            lo, hi = t * 128, (t + 1) * 128
            st = sim[:, lo:hi] + bias[:, lo:hi]
            vt, jt, _ = _fold_rows(st, row_ids, [])
            vs.append(vt)
            js.append(jt)
            norms.append(jnp.max(jnp.where(diag128, sim[lo:hi, lo:hi],
                                           _NEG_BIG),
                                 axis=0, keepdims=True))
        vraw = jnp.concatenate(vs, axis=1)
        j = jnp.concatenate(js, axis=1)
        norm = jnp.concatenate(norms, axis=1)
        best = jnp.maximum(vraw, -1.0)
        sbest_ref[pl.ds(i, 1), pl.ds(bb, 1), :] = best[None]
        sbj_ref[pl.ds(i, 1), pl.ds(bb, 1), :] = (
            (j + (base + bb) * P).astype(jnp.float32)[None])
        snorm_ref[pl.ds(i, 1), pl.ds(bb, 1), :] = norm[None]

    @pl.when(i == nprog - 1)
    def _():
        s = sbest_ref[...].reshape(B, P)
        nm = snorm_ref[...].reshape(B, P)
        bjf = sbj_ref[...].reshape(B, P)
        flatf = (jax.lax.broadcasted_iota(jnp.int32, (B, P), 0) * P
                 + jax.lax.broadcasted_iota(jnp.int32, (B, P), 1)
                 ).astype(jnp.float32)
        lane = jax.lax.broadcasted_iota(jnp.int32, (1, P), 1)
        vsum = jnp.zeros((1, 1), jnp.float32)
        cnt = jnp.zeros((B, P), jnp.float32)
        for _k in range(_GAMMA):
            vw = jnp.max(s, axis=0, keepdims=True)
            v1 = jnp.max(vw, axis=1, keepdims=True)
            i1 = jnp.argmax(vw, axis=1, keepdims=True)
            colf = jnp.where(lane == i1, 1.0, 0.0)
            wmask = jnp.where(s == v1, colf, 0.0)
            s = s + wmask * _NEG_BIG
            cnt = cnt + wmask
            bj1 = jnp.sum(bjf * wmask, keepdims=True)
            cnt = cnt + jnp.where(flatf == bj1, 1.0, 0.0)
            vsum = vsum + v1
        nsum = jnp.sum(nm * cnt, keepdims=True)
        loss_ref[...] = (_LAMBDA_INV / (_GAMMA * D)) * (nsum - 2.0 * vsum)
        cos_ref[...] = vsum / _GAMMA


def kernel(z, view_ids):
    B, P, D = z.shape
    BB = 16
    nprog = B // BB
    vid = view_ids.astype(jnp.int32)
    vidr = vid.reshape(1, P)
    vidc = vid.reshape(P, 1)
    f32 = jnp.float32
    loss2, cos2 = pl.pallas_call(
        functools.partial(_body, bb_per_prog=BB, nprog=nprog, B=B, P=P, D=D),
        grid=(nprog,),
        in_specs=[
            pl.BlockSpec((1, P), lambda i: (0, 0)),
            pl.BlockSpec((P, 1), lambda i: (0, 0)),
            pl.BlockSpec((BB, P, D), lambda i: (i, 0, 0)),
        ],
        out_specs=[
            pl.BlockSpec((1, 1), lambda i: (0, 0)),
            pl.BlockSpec((1, 1), lambda i: (0, 0)),
        ],
        out_shape=[
            jax.ShapeDtypeStruct((1, 1), f32),
            jax.ShapeDtypeStruct((1, 1), f32),
        ],
        scratch_shapes=[
            pltpu.VMEM((nprog, BB, P), f32),
            pltpu.VMEM((nprog, BB, P), f32),
            pltpu.VMEM((nprog, BB, P), f32),
            pltpu.VMEM((P, P), f32),
        ],
        compiler_params=pltpu.CompilerParams(
            dimension_semantics=("arbitrary",),
            vmem_limit_bytes=48 * 1024 * 1024,
        ),
        name="fmatch_sim_fused",
    )(vidr, vidc, z)
    return loss2[0, 0], cos2[0, 0]
